# Initial kernel scaffold; baseline (speedup 1.0000x reference)
import jax
import jax.numpy as jnp
from jax.experimental import pallas as pl


def kernel(keypoint, pc, sn):
    raise NotImplementedError("write your pallas kernel here")


# speedup vs baseline: 1.6764x; 1.6764x over previous
import functools

import jax
import jax.numpy as jnp
from jax import lax
from jax.experimental import pallas as pl
from jax.experimental.pallas import tpu as pltpu
from jax.experimental.pallas import tpu_sc as plsc

_MB = 128
_NC = 2
_NS = 16
_LANES = 16


def _dist_argmin_body(kt_ref, pc_ref, idx_ref, nrm_ref):
    b = pl.program_id(0)
    n = pc_ref.shape[2]
    acc = None
    for c in range(3):
        kc = kt_ref[0, :, c : c + 1]
        pcc = pc_ref[0, c : c + 1, :]
        d = kc - pcc
        acc = d * d if acc is None else acc + d * d
    dmin = jnp.min(acc, axis=1, keepdims=True)
    iota = lax.broadcasted_iota(jnp.int32, acc.shape, 1)
    idx = jnp.min(jnp.where(acc == dmin, iota, n), axis=1)
    idx_ref[0, 0, :] = idx + b * n
    nrm_ref[0, 0, :] = jnp.sqrt(dmin[:, 0])


def _dist_argmin(kt, pc):
    B, M, _ = kt.shape
    N = pc.shape[2]
    grid = (B, M // _MB)
    return pl.pallas_call(
        _dist_argmin_body,
        grid=grid,
        in_specs=[
            pl.BlockSpec((1, _MB, 3), lambda b, j: (b, j, 0)),
            pl.BlockSpec((1, 3, N), lambda b, j: (b, 0, 0)),
        ],
        out_specs=[
            pl.BlockSpec((1, 1, _MB), lambda b, j: (b, 0, j)),
            pl.BlockSpec((1, 1, _MB), lambda b, j: (b, 0, j)),
        ],
        out_shape=[
            jax.ShapeDtypeStruct((B, 1, M), jnp.int32),
            jax.ShapeDtypeStruct((B, 1, M), jnp.float32),
        ],
    )(kt, pc)


def _make_sc_gather_loss(total, wpt):
    mesh = plsc.VectorSubcoreMesh(
        core_axis_name="c", subcore_axis_name="s",
        num_cores=_NC, num_subcores=_NS,
    )

    @functools.partial(
        pl.kernel,
        out_type=jax.ShapeDtypeStruct((total,), jnp.float32),
        mesh=mesh,
        scratch_types=[
            pltpu.VMEM((wpt,), jnp.int32),
            [pltpu.VMEM((wpt,), jnp.float32) for _ in range(6)],
            pltpu.VMEM((wpt,), jnp.float32),
            pltpu.VMEM((wpt,), jnp.float32),
            pltpu.VMEM((wpt,), jnp.float32),
            pltpu.VMEM((wpt,), jnp.float32),
            pltpu.VMEM((wpt,), jnp.float32),
            pltpu.SemaphoreType.DMA,
        ],
    )
    def sc_kernel(p0_hbm, p1_hbm, p2_hbm, s0_hbm, s1_hbm, s2_hbm,
                  idx_hbm, k0_hbm, k1_hbm, k2_hbm, nrm_hbm, out_hbm,
                  idx_v, gat_v, k0_v, k1_v, k2_v, nrm_v, loss_v, sem):
        wid = lax.axis_index("s") * _NC + lax.axis_index("c")
        base = wid * wpt
        pltpu.sync_copy(idx_hbm.at[pl.ds(base, wpt)], idx_v)
        pltpu.sync_copy(k0_hbm.at[pl.ds(base, wpt)], k0_v)
        pltpu.sync_copy(k1_hbm.at[pl.ds(base, wpt)], k1_v)
        pltpu.sync_copy(k2_hbm.at[pl.ds(base, wpt)], k2_v)
        pltpu.sync_copy(nrm_hbm.at[pl.ds(base, wpt)], nrm_v)
        srcs = [p0_hbm, p1_hbm, p2_hbm, s0_hbm, s1_hbm, s2_hbm]
        copies = [
            pltpu.async_copy(src.at[idx_v], dst, sem)
            for src, dst in zip(srcs, gat_v)
        ]
        for cp in copies:
            cp.wait()
        for g in range(wpt // _LANES):
            r0 = g * _LANES
            p0 = gat_v[0][pl.ds(r0, _LANES)]
            p1 = gat_v[1][pl.ds(r0, _LANES)]
            p2 = gat_v[2][pl.ds(r0, _LANES)]
            s0 = gat_v[3][pl.ds(r0, _LANES)]
            s1 = gat_v[4][pl.ds(r0, _LANES)]
            s2 = gat_v[5][pl.ds(r0, _LANES)]
            k0 = k0_v[pl.ds(r0, _LANES)]
            k1 = k1_v[pl.ds(r0, _LANES)]
            k2 = k2_v[pl.ds(r0, _LANES)]
            nrm = nrm_v[pl.ds(r0, _LANES)]
            inv = 1.0 / (nrm + 1e-7)
            t0 = (k0 - p0) * inv
            t1 = (k1 - p1) * inv
            t2 = (k2 - p2) * inv
            dot = s0 * t0 + s1 * t1 + s2 * t2
            loss_v[pl.ds(r0, _LANES)] = dot * dot
        pltpu.sync_copy(loss_v, out_hbm.at[pl.ds(base, wpt)])

    return sc_kernel


def kernel(keypoint, pc, sn):
    B, _, M = keypoint.shape
    N = pc.shape[2]
    kt = jnp.transpose(keypoint, (0, 2, 1))

    idxg, nrm = _dist_argmin(kt, pc)
    idx_flat = idxg.reshape(B * M)
    nrm_flat = nrm.reshape(B * M)

    comps = [pc[:, c, :].reshape(B * N) for c in range(3)]
    comps += [sn[:, c, :].reshape(B * N) for c in range(3)]

    k0 = kt[:, :, 0].reshape(B * M)
    k1 = kt[:, :, 1].reshape(B * M)
    k2 = kt[:, :, 2].reshape(B * M)

    total = B * M
    wpt = total // (_NC * _NS)
    sc_kernel = _make_sc_gather_loss(total, wpt)
    loss = sc_kernel(*comps, idx_flat, k0, k1, k2, nrm_flat)
    return loss.reshape(B, M, 1, 1)


# speedup vs baseline: 1.9097x; 1.1392x over previous
import functools

import jax
import jax.numpy as jnp
from jax import lax
from jax.experimental import pallas as pl
from jax.experimental.pallas import tpu as pltpu
from jax.experimental.pallas import tpu_sc as plsc

_MB = 128
_NCHUNK = 512
_NC = 2
_NS = 16
_LANES = 16


def _dist_argmin_body(kt_ref, pc_ref, idx_ref, nrm_ref):
    b = pl.program_id(0)
    n = pc_ref.shape[2]
    kc = [kt_ref[0, :, c : c + 1] for c in range(3)]
    iota = lax.broadcasted_iota(jnp.int32, (_MB, _NCHUNK), 1)
    run_min = None
    run_idx = None
    for j in range(n // _NCHUNK):
        sl = pl.ds(j * _NCHUNK, _NCHUNK)
        acc = None
        for c in range(3):
            d = kc[c] - pc_ref[0, c : c + 1, sl]
            acc = d * d if acc is None else acc + d * d
        if run_min is None:
            run_min = acc
            run_idx = iota
        else:
            m = acc < run_min
            run_min = jnp.where(m, acc, run_min)
            run_idx = jnp.where(m, iota + (j * _NCHUNK), run_idx)
    gmin = jnp.min(run_min, axis=1, keepdims=True)
    cand = jnp.where(run_min == gmin, run_idx, n)
    idx = jnp.min(cand, axis=1)
    idx_ref[0, 0, :] = idx + (3 * n) * b
    nrm_ref[0, 0, :] = jnp.sqrt(gmin[:, 0])


def _dist_argmin(kt, pc):
    B, M, _ = kt.shape
    N = pc.shape[2]
    grid = (B, M // _MB)
    return pl.pallas_call(
        _dist_argmin_body,
        grid=grid,
        in_specs=[
            pl.BlockSpec((1, _MB, 3), lambda b, j: (b, j, 0)),
            pl.BlockSpec((1, 3, N), lambda b, j: (b, 0, 0)),
        ],
        out_specs=[
            pl.BlockSpec((1, 1, _MB), lambda b, j: (b, 0, j)),
            pl.BlockSpec((1, 1, _MB), lambda b, j: (b, 0, j)),
        ],
        out_shape=[
            jax.ShapeDtypeStruct((B, 1, M), jnp.int32),
            jax.ShapeDtypeStruct((B, 1, M), jnp.float32),
        ],
    )(kt, pc)


def _make_sc_gather_loss(total, wpt, M, N):
    mesh = plsc.VectorSubcoreMesh(
        core_axis_name="c", subcore_axis_name="s",
        num_cores=_NC, num_subcores=_NS,
    )

    @functools.partial(
        pl.kernel,
        out_type=jax.ShapeDtypeStruct((total,), jnp.float32),
        mesh=mesh,
        scratch_types=[
            [pltpu.VMEM((wpt,), jnp.int32) for _ in range(3)],
            [pltpu.VMEM((wpt,), jnp.float32) for _ in range(6)],
            [pltpu.VMEM((wpt,), jnp.float32) for _ in range(3)],
            pltpu.VMEM((wpt,), jnp.float32),
            pltpu.VMEM((wpt,), jnp.float32),
            pltpu.SemaphoreType.DMA,
        ],
    )
    def sc_kernel(pc_hbm, sn_hbm, kp_hbm, idx_hbm, nrm_hbm, out_hbm,
                  idx_v, gat_v, k_v, nrm_v, loss_v, sem):
        wid = lax.axis_index("s") * _NC + lax.axis_index("c")
        base = wid * wpt
        pltpu.sync_copy(idx_hbm.at[pl.ds(base, wpt)], idx_v[0])
        pltpu.sync_copy(nrm_hbm.at[pl.ds(base, wpt)], nrm_v)
        b = base // M
        m0 = base - b * M
        for c in range(3):
            pltpu.sync_copy(kp_hbm.at[pl.ds(b * 3 * M + c * M + m0, wpt)],
                            k_v[c])
        for g in range(wpt // _LANES):
            sl = pl.ds(g * _LANES, _LANES)
            v = idx_v[0][sl]
            idx_v[1][sl] = v + N
            idx_v[2][sl] = v + 2 * N
        copies = [
            pltpu.async_copy(src.at[idx_v[c]], gat_v[3 * s + c], sem)
            for s, src in enumerate((pc_hbm, sn_hbm))
            for c in range(3)
        ]
        for cp in copies:
            cp.wait()
        for g in range(wpt // _LANES):
            sl = pl.ds(g * _LANES, _LANES)
            p0, p1, p2 = gat_v[0][sl], gat_v[1][sl], gat_v[2][sl]
            s0, s1, s2 = gat_v[3][sl], gat_v[4][sl], gat_v[5][sl]
            inv = 1.0 / (nrm_v[sl] + 1e-7)
            t0 = (k_v[0][sl] - p0) * inv
            t1 = (k_v[1][sl] - p1) * inv
            t2 = (k_v[2][sl] - p2) * inv
            dot = s0 * t0 + s1 * t1 + s2 * t2
            loss_v[sl] = dot * dot
        pltpu.sync_copy(loss_v, out_hbm.at[pl.ds(base, wpt)])

    return sc_kernel


def kernel(keypoint, pc, sn):
    B, _, M = keypoint.shape
    N = pc.shape[2]
    kt = jnp.transpose(keypoint, (0, 2, 1))

    idxg, nrm = _dist_argmin(kt, pc)
    idx_flat = idxg.reshape(B * M)
    nrm_flat = nrm.reshape(B * M)

    total = B * M
    wpt = total // (_NC * _NS)
    sc_kernel = _make_sc_gather_loss(total, wpt, M, N)
    loss = sc_kernel(pc.reshape(-1), sn.reshape(-1), keypoint.reshape(-1),
                     idx_flat, nrm_flat)
    return loss.reshape(B, M, 1, 1)
